# baseline (speedup 1.0000x reference)
import functools

import jax
import jax.numpy as jnp
from jax import lax
from jax.experimental import pallas as pl
from jax.experimental.pallas import tpu as pltpu
from jax.experimental.pallas import tpu_sc as plsc

VOCAB = 100000
D = 256
N_MID = 128
N_OUT = 2
B = 4096
L = 50
LANES = 16
CHUNKS = D // LANES
HALF = 128


def _sc_pool2(emb2, idx3):
    info = plsc.get_sparse_core_info()
    NC, NS = info.num_cores, info.num_subcores
    NW = NC * NS
    S_PER_W = B // NW
    PAIRS = S_PER_W // 2
    SEG = 104

    mesh = plsc.VectorSubcoreMesh(core_axis_name="c", subcore_axis_name="s")

    @functools.partial(
        pl.kernel,
        mesh=mesh,
        out_type=jax.ShapeDtypeStruct((B, D), jnp.float32),
        scratch_types=[
            pltpu.VMEM((PAIRS, 2, SEG), jnp.int32),
            pltpu.VMEM((2, 256, HALF), jnp.float32),
            pltpu.VMEM((S_PER_W, D), jnp.float32),
            pltpu.SemaphoreType.DMA,
            pltpu.SemaphoreType.DMA,
        ],
    )
    def pool(emb_hbm, idx_hbm, out_hbm, idx_v, rows_v, acc_v, sem0, sem1):
        wid = lax.axis_index("s") * NC + lax.axis_index("c")
        pltpu.sync_copy(idx_hbm.at[wid], idx_v)
        sems = (sem0, sem1)

        def start(r, b):
            pltpu.make_async_copy(
                emb_hbm.at[idx_v.at[r, 0]], rows_v.at[b, pl.ds(0, SEG)],
                sems[b],
            ).start()
            pltpu.make_async_copy(
                emb_hbm.at[idx_v.at[r, 1]], rows_v.at[b, pl.ds(SEG, SEG)],
                sems[b],
            ).start()

        def wait(b):
            for _ in range(2):
                pltpu.make_async_copy(
                    emb_hbm.at[idx_v.at[0, 0]], rows_v.at[b, pl.ds(0, SEG)],
                    sems[b],
                ).wait()

        def accum(r, b):
            for s in range(2):
                def jbody(j, carry):
                    r2 = 2 * (s * L + j)
                    out = []
                    for c in range(CHUNKS):
                        h, cc = divmod(c, 8)
                        out.append(
                            carry[c]
                            + rows_v[b, r2 + h, pl.ds(cc * LANES, LANES)]
                        )
                    return tuple(out)

                init = []
                for c in range(CHUNKS):
                    h, cc = divmod(c, 8)
                    init.append(rows_v[b, 2 * s * L + h, pl.ds(cc * LANES, LANES)])
                acc = lax.fori_loop(1, L, jbody, tuple(init))
                samp = 2 * r + s
                for c in range(CHUNKS):
                    acc_v[samp, pl.ds(c * LANES, LANES)] = acc[c]

        start(0, 0)

        def outer(i, carry):
            for b in range(2):
                r = 2 * i + b

                @pl.when(r + 1 < PAIRS)
                def _():
                    start(r + 1, 1 - b)

                wait(b)
                accum(r, b)
            return carry

        lax.fori_loop(0, PAIRS // 2, outer, 0)
        pltpu.sync_copy(acc_v, out_hbm.at[pl.ds(wid * S_PER_W, S_PER_W)])

    return pool(emb2, idx3)


def _mlp_tc(x2, mask, W1, b1, W2, b2):
    BLK = 512

    def body(x2_ref, mask_ref, w1_ref, b1_ref, w2_ref, b2_ref, out_ref):
        m = jnp.sum(mask_ref[...], axis=1, keepdims=True)
        x3 = x2_ref[...] / m
        h1 = jnp.tanh(
            lax.dot_general(
                x3, w1_ref[...], (((1,), (1,)), ((), ())),
                preferred_element_type=jnp.float32,
                precision=lax.Precision.HIGHEST,
            )
            + b1_ref[...]
        )
        out_ref[...] = (
            lax.dot_general(
                h1, w2_ref[...], (((1,), (1,)), ((), ())),
                preferred_element_type=jnp.float32,
                precision=lax.Precision.HIGHEST,
            )
            + b2_ref[...]
        )

    return pl.pallas_call(
        body,
        grid=(B // BLK,),
        in_specs=[
            pl.BlockSpec((BLK, D), lambda i: (i, 0)),
            pl.BlockSpec((BLK, L), lambda i: (i, 0)),
            pl.BlockSpec((N_MID, D), lambda i: (0, 0)),
            pl.BlockSpec((1, N_MID), lambda i: (0, 0)),
            pl.BlockSpec((N_OUT, N_MID), lambda i: (0, 0)),
            pl.BlockSpec((1, N_OUT), lambda i: (0, 0)),
        ],
        out_specs=pl.BlockSpec((BLK, N_OUT), lambda i: (i, 0)),
        out_shape=jax.ShapeDtypeStruct((B, N_OUT), jnp.float32),
    )(x2, mask, W1, b1.reshape(1, N_MID), W2, b2.reshape(1, N_OUT))


def kernel(x, mask, emb, W1, b1, W2, b2):
    info = plsc.get_sparse_core_info()
    NW = info.num_cores * info.num_subcores
    PAIRS = (B // NW) // 2
    idx3 = x.astype(jnp.int32).reshape(NW, PAIRS, 2 * L)
    pad = (
        jnp.arange(NW * PAIRS * 4, dtype=jnp.int32).reshape(NW, PAIRS, 4)
        * 97
    ) % VOCAB
    idx3 = jnp.concatenate([idx3, pad], axis=2)
    idx6 = jnp.stack([2 * idx3, 2 * idx3 + 1], axis=-1)
    idx6 = idx6.reshape(NW, PAIRS, 2, 104)
    emb2 = emb.reshape(2 * VOCAB, HALF)
    x2 = _sc_pool2(emb2, idx6)
    return _mlp_tc(x2, mask, W1, b1, W2, b2)


# speedup vs baseline: 1.8120x; 1.8120x over previous
import functools

import jax
import jax.numpy as jnp
from jax import lax
from jax.experimental import pallas as pl
from jax.experimental.pallas import tpu as pltpu
from jax.experimental.pallas import tpu_sc as plsc

VOCAB = 100000
D = 256
N_MID = 128
N_OUT = 2
B = 4096
L = 50
LANES = 16
CHUNKS = D // LANES


def _sc_pool(emb, idx3):
    info = plsc.get_sparse_core_info()
    NC, NS = info.num_cores, info.num_subcores
    NW = NC * NS
    S_PER_W = B // NW
    PAIRS = S_PER_W // 2
    ROWS = 2 * L
    RPAD = 104

    mesh = plsc.VectorSubcoreMesh(core_axis_name="c", subcore_axis_name="s")

    @functools.partial(
        pl.kernel,
        mesh=mesh,
        out_type=jax.ShapeDtypeStruct((B, D), jnp.float32),
        scratch_types=[
            pltpu.VMEM((PAIRS, RPAD), jnp.int32),
            pltpu.VMEM((2, 128, D), jnp.float32),
            pltpu.VMEM((S_PER_W, D), jnp.float32),
            pltpu.SemaphoreType.DMA,
            pltpu.SemaphoreType.DMA,
        ],
    )
    def pool(emb_hbm, idx_hbm, out_hbm, idx_v, rows_v, acc_v, sem0, sem1):
        wid = lax.axis_index("s") * NC + lax.axis_index("c")
        pltpu.sync_copy(idx_hbm.at[wid], idx_v)
        sems = (sem0, sem1)

        def start(r, b):
            pltpu.make_async_copy(
                emb_hbm.at[idx_v.at[r]], rows_v.at[b, pl.ds(0, RPAD)], sems[b]
            ).start()

        def wait(b):
            pltpu.make_async_copy(
                emb_hbm.at[idx_v.at[0]], rows_v.at[b, pl.ds(0, RPAD)], sems[b]
            ).wait()

        def accum(r, b):
            for s in range(2):
                def jbody(j, carry):
                    row = s * L + j
                    return tuple(
                        carry[c] + rows_v[b, row, pl.ds(c * LANES, LANES)]
                        for c in range(CHUNKS)
                    )
                init = tuple(
                    rows_v[b, s * L, pl.ds(c * LANES, LANES)]
                    for c in range(CHUNKS)
                )
                acc = lax.fori_loop(1, L, jbody, init)
                samp = 2 * r + s
                for c in range(CHUNKS):
                    acc_v[samp, pl.ds(c * LANES, LANES)] = acc[c]

        start(0, 0)

        def outer(i, carry):
            for b in range(2):
                r = 2 * i + b

                @pl.when(r + 1 < PAIRS)
                def _():
                    start(r + 1, 1 - b)

                wait(b)
                accum(r, b)
            return carry

        lax.fori_loop(0, PAIRS // 2, outer, 0)
        pltpu.sync_copy(acc_v, out_hbm.at[pl.ds(wid * S_PER_W, S_PER_W)])

    return pool(emb, idx3)


def _mlp_tc(x2, mask, W1, b1, W2, b2):
    BLK = 1024

    def body(x2_ref, mask_ref, w1_ref, b1_ref, w2_ref, b2_ref, out_ref):
        m = jnp.sum(mask_ref[...], axis=1, keepdims=True)
        x3 = x2_ref[...] / m
        h1 = jnp.tanh(
            lax.dot_general(
                x3, w1_ref[...], (((1,), (1,)), ((), ())),
                preferred_element_type=jnp.float32,
                precision=lax.Precision.HIGHEST,
            )
            + b1_ref[...]
        )
        out_ref[...] = (
            lax.dot_general(
                h1, w2_ref[...], (((1,), (1,)), ((), ())),
                preferred_element_type=jnp.float32,
                precision=lax.Precision.HIGHEST,
            )
            + b2_ref[...]
        )

    return pl.pallas_call(
        body,
        grid=(B // BLK,),
        in_specs=[
            pl.BlockSpec((BLK, D), lambda i: (i, 0)),
            pl.BlockSpec((BLK, L), lambda i: (i, 0)),
            pl.BlockSpec((N_MID, D), lambda i: (0, 0)),
            pl.BlockSpec((1, N_MID), lambda i: (0, 0)),
            pl.BlockSpec((N_OUT, N_MID), lambda i: (0, 0)),
            pl.BlockSpec((1, N_OUT), lambda i: (0, 0)),
        ],
        out_specs=pl.BlockSpec((BLK, N_OUT), lambda i: (i, 0)),
        out_shape=jax.ShapeDtypeStruct((B, N_OUT), jnp.float32),
    )(x2, mask, W1, b1.reshape(1, N_MID), W2, b2.reshape(1, N_OUT))


def kernel(x, mask, emb, W1, b1, W2, b2):
    info = plsc.get_sparse_core_info()
    NW = info.num_cores * info.num_subcores
    idx3 = x.astype(jnp.int32).reshape(NW, (B // NW) // 2, 2 * L)
    PAIRS = (B // NW) // 2
    pad = (
        jnp.arange(NW * PAIRS * 4, dtype=jnp.int32).reshape(NW, PAIRS, 4)
        * 97
    ) % VOCAB
    idx3 = jnp.concatenate([idx3, pad], axis=2)
    x2 = _sc_pool(emb, idx3)
    return _mlp_tc(x2, mask, W1, b1, W2, b2)
